# baseline (speedup 1.0000x reference)
import functools

import jax
import jax.numpy as jnp
from jax import lax
from jax.experimental import pallas as pl
from jax.experimental.pallas import tpu as pltpu
from jax.experimental.pallas import tpu_sc as plsc

BATCH = 16384
NF = 64

_info = plsc.get_sparse_core_info()
_NC, _NS = _info.num_cores, _info.num_subcores
_NW = _NC * _NS
_BPW = BATCH // _NW


def _repack_body(in_ref, out_ref):
    c = pl.program_id(1)

    @pl.when(c == 0)
    def _():
        out_ref[:, :NF] = in_ref[:]

    @pl.when(c == 1)
    def _():
        out_ref[:, NF:] = in_ref[:]


def _repack(table, block_rows):
    n = table.shape[0]
    half = n // 2
    assert half % block_rows == 0 and block_rows % 8 == 0
    g = half // block_rows
    return pl.pallas_call(
        _repack_body,
        grid=(g, 2),
        in_specs=[pl.BlockSpec((block_rows, NF), lambda i, c, g=g: (i + c * g, 0))],
        out_specs=pl.BlockSpec((block_rows, 2 * NF), lambda i, c: (i, 0)),
        out_shape=jax.ShapeDtypeStruct((half, 2 * NF), jnp.float32),
    )(table)


def _gather_body(half, tab_hbm, src_hbm, out_hbm, idx_v, tix_v, rows_v, sem):
    wid = lax.axis_index("s") * _NC + lax.axis_index("c")
    base = wid * _BPW
    pltpu.sync_copy(src_hbm.at[pl.ds(base, _BPW)], idx_v)
    for k in range(_BPW // 16):
        iv = idx_v[pl.ds(k * 16, 16)]
        wrap = jnp.where(iv >= half, half, 0)
        tix_v[pl.ds(k * 16, 16)] = iv - wrap
    pltpu.async_copy(tab_hbm.at[tix_v], rows_v, sem).wait()
    pltpu.sync_copy(rows_v, out_hbm.at[pl.ds(base, _BPW)])


def _make_sc_gather(half):
    return functools.partial(
        pl.kernel,
        out_type=jax.ShapeDtypeStruct((BATCH, 2 * NF), jnp.float32),
        mesh=plsc.VectorSubcoreMesh(core_axis_name="c", subcore_axis_name="s"),
        scratch_types=[
            pltpu.VMEM((_BPW,), jnp.int32),
            pltpu.VMEM((_BPW,), jnp.int32),
            pltpu.VMEM((_BPW, 2 * NF), jnp.float32),
            pltpu.SemaphoreType.DMA,
        ],
    )(functools.partial(_gather_body, half))


_sc_gather_u = _make_sc_gather(500000)
_sc_gather_m = _make_sc_gather(50000)


def _mlp_body(eu_ref, em_ref, pu_ref, pm_ref, w1u_ref, w1m_ref, b1_ref,
              w2_ref, b2_ref, out_ref):
    eu = eu_ref[:]
    em = em_ref[:]
    au = jnp.dot(eu[:, :NF], w1u_ref[:], preferred_element_type=jnp.float32)
    bu = jnp.dot(eu[:, NF:], w1u_ref[:], preferred_element_type=jnp.float32)
    am = jnp.dot(em[:, :NF], w1m_ref[:], preferred_element_type=jnp.float32)
    bm = jnp.dot(em[:, NF:], w1m_ref[:], preferred_element_type=jnp.float32)
    hu = jnp.where(pu_ref[:] > 0, bu, au)
    hm = jnp.where(pm_ref[:] > 0, bm, am)
    h = jnp.maximum(hu + hm + b1_ref[:], 0.0)
    o = jnp.dot(h, w2_ref[:], preferred_element_type=jnp.float32) + b2_ref[:]
    out_ref[:] = jax.nn.sigmoid(o) * 6.0 - 0.5


def kernel(users, movies, U, M, W1, b1, W2, b2):
    users = users.astype(jnp.int32)
    movies = movies.astype(jnp.int32)
    Mp = _repack(M, 5000)
    em = _sc_gather_m(Mp, movies)
    Up = _repack(U, 5000)
    eu = _sc_gather_u(Up, users)
    pu = (users >= 500000)[:, None].astype(jnp.int32)
    pm = (movies >= 50000)[:, None].astype(jnp.int32)
    w1u = W1[:, :NF].T
    w1m = W1[:, NF:].T
    blk = 4096
    out2d = pl.pallas_call(
        _mlp_body,
        grid=(BATCH // blk,),
        in_specs=[
            pl.BlockSpec((blk, 2 * NF), lambda i: (i, 0)),
            pl.BlockSpec((blk, 2 * NF), lambda i: (i, 0)),
            pl.BlockSpec((blk, 1), lambda i: (i, 0)),
            pl.BlockSpec((blk, 1), lambda i: (i, 0)),
            pl.BlockSpec((NF, 10), lambda i: (0, 0)),
            pl.BlockSpec((NF, 10), lambda i: (0, 0)),
            pl.BlockSpec((1, 10), lambda i: (0, 0)),
            pl.BlockSpec((10, 1), lambda i: (0, 0)),
            pl.BlockSpec((1, 1), lambda i: (0, 0)),
        ],
        out_specs=pl.BlockSpec((blk, 1), lambda i: (i, 0)),
        out_shape=jax.ShapeDtypeStruct((BATCH, 1), jnp.float32),
    )(eu, em, pu, pm, w1u, w1m, b1[None, :], W2.T, b2[None, :])
    return out2d[:, 0]


# speedup vs baseline: 4.6356x; 4.6356x over previous
import functools

import jax
import jax.numpy as jnp
from jax import lax
from jax.experimental import pallas as pl
from jax.experimental.pallas import tpu as pltpu
from jax.experimental.pallas import tpu_sc as plsc

BATCH = 16384
NF = 64
HP = 16

_info = plsc.get_sparse_core_info()
_NC, _NS = _info.num_cores, _info.num_subcores
_NW = _NC * _NS
_BPW = BATCH // _NW


def _project_body(w_ref, t_ref, *out_refs):
    p = jnp.dot(w_ref[:], t_ref[:], preferred_element_type=jnp.float32)
    for f, o_ref in enumerate(out_refs):
        o_ref[:] = p[f, :]


def _project(w16, table_t, cols):
    n = table_t.shape[1]
    grid = pl.cdiv(n, cols)
    return pl.pallas_call(
        _project_body,
        grid=(grid,),
        in_specs=[
            pl.BlockSpec((HP, NF), lambda i: (0, 0)),
            pl.BlockSpec((NF, cols), lambda i: (0, i)),
        ],
        out_specs=[pl.BlockSpec((cols,), lambda i: (i,)) for _ in range(HP)],
        out_shape=[jax.ShapeDtypeStruct((n,), jnp.float32) for _ in range(HP)],
    )(w16, table_t)


def _tail_body(users_hbm, movies_hbm, w2_hbm, b1_hbm, b2_hbm, *rest):
    pu_hbm = rest[:HP]
    pm_hbm = rest[HP:2 * HP]
    out_hbm = rest[2 * HP]
    idx_u, idx_m, coef_v, acc_v, gath = rest[2 * HP + 1:2 * HP + 6]
    sem = rest[2 * HP + 6]

    wid = lax.axis_index("s") * _NC + lax.axis_index("c")
    base = wid * _BPW
    pltpu.sync_copy(users_hbm.at[pl.ds(base, _BPW)], idx_u)
    pltpu.sync_copy(movies_hbm.at[pl.ds(base, _BPW)], idx_m)
    pltpu.sync_copy(w2_hbm, coef_v.at[0])
    pltpu.sync_copy(b1_hbm, coef_v.at[1])
    pltpu.sync_copy(b2_hbm, coef_v.at[2, pl.ds(0, 1)])

    copies = []
    for f in range(HP):
        copies.append(pltpu.async_copy(pu_hbm[f].at[idx_u], gath.at[0, f], sem))
        copies.append(pltpu.async_copy(pm_hbm[f].at[idx_m], gath.at[1, f], sem))
    for c in copies:
        c.wait()

    w2v = coef_v[0]
    b1v = coef_v[1]
    b2v = coef_v[2]

    def group(g, carry):
        sl = pl.ds(g * 16, 16)
        acc = jnp.zeros((16,), jnp.float32) + b2v[0]
        for f in range(HP):
            h = gath[0, f, sl] + gath[1, f, sl] + b1v[f]
            acc = acc + w2v[f] * jnp.maximum(h, 0.0)
        sig = 1.0 / (1.0 + jnp.exp(-acc))
        acc_v[sl] = sig * 6.0 - 0.5
        return carry

    lax.fori_loop(0, _BPW // 16, group, 0)
    pltpu.sync_copy(acc_v, out_hbm.at[pl.ds(base, _BPW)])


_sc_tail = functools.partial(
    pl.kernel,
    out_type=jax.ShapeDtypeStruct((BATCH,), jnp.float32),
    mesh=plsc.VectorSubcoreMesh(core_axis_name="c", subcore_axis_name="s"),
    compiler_params=pltpu.CompilerParams(use_tc_tiling_on_sc=False),
    scratch_types=[
        pltpu.VMEM((_BPW,), jnp.int32),
        pltpu.VMEM((_BPW,), jnp.int32),
        pltpu.VMEM((3, HP), jnp.float32),
        pltpu.VMEM((_BPW,), jnp.float32),
        pltpu.VMEM((2, HP, _BPW), jnp.float32),
        pltpu.SemaphoreType.DMA,
    ],
)(_tail_body)


def kernel(users, movies, U, M, W1, b1, W2, b2):
    users = users.astype(jnp.int32)
    movies = movies.astype(jnp.int32)
    w1u = jnp.zeros((HP, NF), jnp.float32).at[:10].set(W1[:, :NF])
    w1m = jnp.zeros((HP, NF), jnp.float32).at[:10].set(W1[:, NF:])
    w2p = jnp.zeros((HP,), jnp.float32).at[:10].set(W2[0])
    b1p = jnp.zeros((HP,), jnp.float32).at[:10].set(b1)
    pu = _project(w1u, U.T, 16384)
    pm = _project(w1m, M.T, 4096)
    out = _sc_tail(users, movies, w2p, b1p, b2, *pu, *pm)
    return out


# speedup vs baseline: 5.3247x; 1.1487x over previous
import functools

import jax
import jax.numpy as jnp
from jax import lax
from jax.experimental import pallas as pl
from jax.experimental.pallas import tpu as pltpu
from jax.experimental.pallas import tpu_sc as plsc

BATCH = 16384
NF = 64
HP = 16

_info = plsc.get_sparse_core_info()
_NC, _NS = _info.num_cores, _info.num_subcores
_NW = _NC * _NS
_BPW = BATCH // _NW


def _project_body(w_ref, t_ref, *out_refs):
    p = jnp.dot(w_ref[:], t_ref[:], preferred_element_type=jnp.float32)
    for f, o_ref in enumerate(out_refs):
        o_ref[:] = p[f, :]


def _project(w16, table_t, cols):
    n = table_t.shape[1]
    grid = pl.cdiv(n, cols)
    return pl.pallas_call(
        _project_body,
        grid=(grid,),
        in_specs=[
            pl.BlockSpec((HP, NF), lambda i: (0, 0)),
            pl.BlockSpec((NF, cols), lambda i: (0, i)),
        ],
        out_specs=[pl.BlockSpec((cols,), lambda i: (i,)) for _ in range(HP)],
        out_shape=[jax.ShapeDtypeStruct((n,), jnp.float32) for _ in range(HP)],
    )(w16, table_t)


def _tail_body(users_hbm, movies_hbm, w2_hbm, b1_hbm, b2_hbm, *rest):
    pu_hbm = rest[:HP]
    pm_hbm = rest[HP:2 * HP]
    out_hbm = rest[2 * HP]
    idx_u, idx_m, coef_v, acc_v, gath = rest[2 * HP + 1:2 * HP + 6]
    sem = rest[2 * HP + 6]

    wid = lax.axis_index("s") * _NC + lax.axis_index("c")
    base = wid * _BPW
    pltpu.sync_copy(users_hbm.at[pl.ds(base, _BPW)], idx_u)
    pltpu.sync_copy(movies_hbm.at[pl.ds(base, _BPW)], idx_m)
    pltpu.sync_copy(w2_hbm, coef_v.at[0])
    pltpu.sync_copy(b1_hbm, coef_v.at[1])
    pltpu.sync_copy(b2_hbm, coef_v.at[2, pl.ds(0, 1)])

    copies = []
    for f in range(HP):
        copies.append(pltpu.async_copy(pu_hbm[f].at[idx_u], gath.at[0, f], sem))
        copies.append(pltpu.async_copy(pm_hbm[f].at[idx_m], gath.at[1, f], sem))
    for c in copies:
        c.wait()

    w2v = coef_v[0]
    b1v = coef_v[1]
    b2v = coef_v[2]

    def group(g, carry):
        sl = pl.ds(g * 16, 16)
        acc = jnp.zeros((16,), jnp.float32) + b2v[0]
        for f in range(HP):
            h = gath[0, f, sl] + gath[1, f, sl] + b1v[f]
            acc = acc + w2v[f] * jnp.maximum(h, 0.0)
        sig = 1.0 / (1.0 + jnp.exp(-acc))
        acc_v[sl] = sig * 6.0 - 0.5
        return carry

    lax.fori_loop(0, _BPW // 16, group, 0)
    pltpu.sync_copy(acc_v, out_hbm.at[pl.ds(base, _BPW)])


_sc_tail = functools.partial(
    pl.kernel,
    out_type=jax.ShapeDtypeStruct((BATCH,), jnp.float32),
    mesh=plsc.VectorSubcoreMesh(core_axis_name="c", subcore_axis_name="s"),
    compiler_params=pltpu.CompilerParams(use_tc_tiling_on_sc=False),
    scratch_types=[
        pltpu.VMEM((_BPW,), jnp.int32),
        pltpu.VMEM((_BPW,), jnp.int32),
        pltpu.VMEM((3, HP), jnp.float32),
        pltpu.VMEM((_BPW,), jnp.float32),
        pltpu.VMEM((2, HP, _BPW), jnp.float32),
        pltpu.SemaphoreType.DMA,
    ],
)(_tail_body)


def kernel(users, movies, U, M, W1, b1, W2, b2):
    users = users.astype(jnp.int32)
    movies = movies.astype(jnp.int32)
    w1u = jnp.zeros((HP, NF), jnp.float32).at[:10].set(W1[:, :NF])
    w1m = jnp.zeros((HP, NF), jnp.float32).at[:10].set(W1[:, NF:])
    w2p = jnp.zeros((HP,), jnp.float32).at[:10].set(W2[0])
    b1p = jnp.zeros((HP,), jnp.float32).at[:10].set(b1)
    pu = _project(w1u, U.T, 32768)
    pm = _project(w1m, M.T, 16384)
    out = _sc_tail(users, movies, w2p, b1p, b2, *pu, *pm)
    return out


# speedup vs baseline: 5.4709x; 1.0275x over previous
import functools

import jax
import jax.numpy as jnp
from jax import lax
from jax.experimental import pallas as pl
from jax.experimental.pallas import tpu as pltpu
from jax.experimental.pallas import tpu_sc as plsc

BATCH = 16384
NF = 64
HP = 16

_info = plsc.get_sparse_core_info()
_NC, _NS = _info.num_cores, _info.num_subcores
_NW = _NC * _NS
_BPW = BATCH // _NW


def _project_body(w_ref, t_ref, *out_refs):
    p = jnp.dot(w_ref[:], t_ref[:], preferred_element_type=jnp.float32)
    for f, o_ref in enumerate(out_refs):
        o_ref[:] = p[f, :]


def _project(w16, table_t, cols):
    n = table_t.shape[1]
    grid = pl.cdiv(n, cols)
    return pl.pallas_call(
        _project_body,
        grid=(grid,),
        in_specs=[
            pl.BlockSpec((HP, NF), lambda i: (0, 0)),
            pl.BlockSpec((NF, cols), lambda i: (0, i)),
        ],
        out_specs=[pl.BlockSpec((cols,), lambda i: (i,)) for _ in range(HP)],
        out_shape=[jax.ShapeDtypeStruct((n,), jnp.float32) for _ in range(HP)],
    )(w16, table_t)


def _mgather_body(movies_hbm, *rest):
    pm_hbm = rest[:HP]
    out_hbm = rest[HP:2 * HP]
    idx_m, gath, sem = rest[2 * HP:2 * HP + 3]

    wid = lax.axis_index("s") * _NC + lax.axis_index("c")
    base = wid * _BPW
    pltpu.sync_copy(movies_hbm.at[pl.ds(base, _BPW)], idx_m)
    copies = [pltpu.async_copy(pm_hbm[f].at[idx_m], gath.at[f], sem)
              for f in range(HP)]
    for c in copies:
        c.wait()
    for f in range(HP):
        pltpu.sync_copy(gath.at[f], out_hbm[f].at[pl.ds(base, _BPW)])


_sc_mgather = functools.partial(
    pl.kernel,
    out_type=[jax.ShapeDtypeStruct((BATCH,), jnp.float32) for _ in range(HP)],
    mesh=plsc.VectorSubcoreMesh(core_axis_name="c", subcore_axis_name="s"),
    compiler_params=pltpu.CompilerParams(use_tc_tiling_on_sc=False),
    scratch_types=[
        pltpu.VMEM((_BPW,), jnp.int32),
        pltpu.VMEM((HP, _BPW), jnp.float32),
        pltpu.SemaphoreType.DMA,
    ],
)(_mgather_body)


def _tail_body(users_hbm, w2_hbm, b1_hbm, b2_hbm, *rest):
    pu_hbm = rest[:HP]
    hm_hbm = rest[HP:2 * HP]
    out_hbm = rest[2 * HP]
    idx_u, coef_v, acc_v, gath = rest[2 * HP + 1:2 * HP + 5]
    sem = rest[2 * HP + 5]

    wid = lax.axis_index("s") * _NC + lax.axis_index("c")
    base = wid * _BPW
    pltpu.sync_copy(users_hbm.at[pl.ds(base, _BPW)], idx_u)
    pltpu.sync_copy(w2_hbm, coef_v.at[0])
    pltpu.sync_copy(b1_hbm, coef_v.at[1])
    pltpu.sync_copy(b2_hbm, coef_v.at[2, pl.ds(0, 1)])

    copies = [pltpu.async_copy(pu_hbm[f].at[idx_u], gath.at[0, f], sem)
              for f in range(HP)]
    copies += [pltpu.async_copy(hm_hbm[f].at[pl.ds(base, _BPW)], gath.at[1, f], sem)
               for f in range(HP)]
    for c in copies:
        c.wait()

    w2v = coef_v[0]
    b1v = coef_v[1]
    b2v = coef_v[2]

    def group(g, carry):
        sl = pl.ds(g * 16, 16)
        acc = jnp.zeros((16,), jnp.float32) + b2v[0]
        for f in range(HP):
            h = gath[0, f, sl] + gath[1, f, sl] + b1v[f]
            acc = acc + w2v[f] * jnp.maximum(h, 0.0)
        sig = 1.0 / (1.0 + jnp.exp(-acc))
        acc_v[sl] = sig * 6.0 - 0.5
        return carry

    lax.fori_loop(0, _BPW // 16, group, 0)
    pltpu.sync_copy(acc_v, out_hbm.at[pl.ds(base, _BPW)])


_sc_tail = functools.partial(
    pl.kernel,
    out_type=jax.ShapeDtypeStruct((BATCH,), jnp.float32),
    mesh=plsc.VectorSubcoreMesh(core_axis_name="c", subcore_axis_name="s"),
    compiler_params=pltpu.CompilerParams(use_tc_tiling_on_sc=False),
    scratch_types=[
        pltpu.VMEM((_BPW,), jnp.int32),
        pltpu.VMEM((3, HP), jnp.float32),
        pltpu.VMEM((_BPW,), jnp.float32),
        pltpu.VMEM((2, HP, _BPW), jnp.float32),
        pltpu.SemaphoreType.DMA,
    ],
)(_tail_body)


def kernel(users, movies, U, M, W1, b1, W2, b2):
    users = users.astype(jnp.int32)
    movies = movies.astype(jnp.int32)
    w1u = jnp.zeros((HP, NF), jnp.float32).at[:10].set(W1[:, :NF])
    w1m = jnp.zeros((HP, NF), jnp.float32).at[:10].set(W1[:, NF:])
    w2p = jnp.zeros((HP,), jnp.float32).at[:10].set(W2[0])
    b1p = jnp.zeros((HP,), jnp.float32).at[:10].set(b1)
    pm = _project(w1m, M.T, 16384)
    hm = _sc_mgather(movies, *pm)
    pu = _project(w1u, U.T, 32768)
    out = _sc_tail(users, w2p, b1p, b2, *pu, *hm)
    return out


# speedup vs baseline: 6.0728x; 1.1100x over previous
import functools

import jax
import jax.numpy as jnp
from jax import lax
from jax.experimental import pallas as pl
from jax.experimental.pallas import tpu as pltpu
from jax.experimental.pallas import tpu_sc as plsc

BATCH = 16384
NF = 64
HP = 16
NH = 10

_info = plsc.get_sparse_core_info()
_NC, _NS = _info.num_cores, _info.num_subcores
_NW = _NC * _NS
_BPW = BATCH // _NW


def _project_body(w_ref, t_ref, *out_refs):
    p = jnp.dot(w_ref[:], t_ref[:], preferred_element_type=jnp.float32)
    for f, o_ref in enumerate(out_refs):
        o_ref[:] = p[f, :]


def _project(w16, table_t, cols):
    n = table_t.shape[1]
    grid = pl.cdiv(n, cols)
    return pl.pallas_call(
        _project_body,
        grid=(grid,),
        in_specs=[
            pl.BlockSpec((HP, NF), lambda i: (0, 0)),
            pl.BlockSpec((NF, cols), lambda i: (0, i)),
        ],
        out_specs=[pl.BlockSpec((cols,), lambda i: (i,)) for _ in range(NH)],
        out_shape=[jax.ShapeDtypeStruct((n,), jnp.float32) for _ in range(NH)],
    )(w16, table_t)


def _mgather_body(movies_hbm, *rest):
    pm_hbm = rest[:NH]
    out_hbm = rest[NH:2 * NH]
    idx_m, gath, sem = rest[2 * NH:2 * NH + 3]

    wid = lax.axis_index("s") * _NC + lax.axis_index("c")
    base = wid * _BPW
    pltpu.sync_copy(movies_hbm.at[pl.ds(base, _BPW)], idx_m)
    copies = [pltpu.async_copy(pm_hbm[f].at[idx_m], gath.at[f], sem)
              for f in range(NH)]
    for c in copies:
        c.wait()
    for f in range(NH):
        pltpu.sync_copy(gath.at[f], out_hbm[f].at[pl.ds(base, _BPW)])


_sc_mgather = functools.partial(
    pl.kernel,
    out_type=[jax.ShapeDtypeStruct((BATCH,), jnp.float32) for _ in range(NH)],
    mesh=plsc.VectorSubcoreMesh(core_axis_name="c", subcore_axis_name="s"),
    compiler_params=pltpu.CompilerParams(use_tc_tiling_on_sc=False),
    scratch_types=[
        pltpu.VMEM((_BPW,), jnp.int32),
        pltpu.VMEM((NH, _BPW), jnp.float32),
        pltpu.SemaphoreType.DMA,
    ],
)(_mgather_body)


def _tail_body(users_hbm, w2_hbm, b1_hbm, b2_hbm, *rest):
    pu_hbm = rest[:NH]
    hm_hbm = rest[NH:2 * NH]
    out_hbm = rest[2 * NH]
    idx_u, coef_v, acc_v, gath = rest[2 * NH + 1:2 * NH + 5]
    sem = rest[2 * NH + 5]

    wid = lax.axis_index("s") * _NC + lax.axis_index("c")
    base = wid * _BPW
    pltpu.sync_copy(users_hbm.at[pl.ds(base, _BPW)], idx_u)
    pltpu.sync_copy(w2_hbm, coef_v.at[0])
    pltpu.sync_copy(b1_hbm, coef_v.at[1])
    pltpu.sync_copy(b2_hbm, coef_v.at[2, pl.ds(0, 1)])

    copies = [pltpu.async_copy(pu_hbm[f].at[idx_u], gath.at[0, f], sem)
              for f in range(NH)]
    copies += [pltpu.async_copy(hm_hbm[f].at[pl.ds(base, _BPW)], gath.at[1, f], sem)
               for f in range(NH)]
    for c in copies:
        c.wait()

    w2v = coef_v[0]
    b1v = coef_v[1]
    b2v = coef_v[2]

    def group(g, carry):
        sl = pl.ds(g * 16, 16)
        acc = jnp.zeros((16,), jnp.float32) + b2v[0]
        for f in range(NH):
            h = gath[0, f, sl] + gath[1, f, sl] + b1v[f]
            acc = acc + w2v[f] * jnp.maximum(h, 0.0)
        sig = 1.0 / (1.0 + jnp.exp(-acc))
        acc_v[sl] = sig * 6.0 - 0.5
        return carry

    lax.fori_loop(0, _BPW // 16, group, 0)
    pltpu.sync_copy(acc_v, out_hbm.at[pl.ds(base, _BPW)])


_sc_tail = functools.partial(
    pl.kernel,
    out_type=jax.ShapeDtypeStruct((BATCH,), jnp.float32),
    mesh=plsc.VectorSubcoreMesh(core_axis_name="c", subcore_axis_name="s"),
    compiler_params=pltpu.CompilerParams(use_tc_tiling_on_sc=False),
    scratch_types=[
        pltpu.VMEM((_BPW,), jnp.int32),
        pltpu.VMEM((3, HP), jnp.float32),
        pltpu.VMEM((_BPW,), jnp.float32),
        pltpu.VMEM((2, NH, _BPW), jnp.float32),
        pltpu.SemaphoreType.DMA,
    ],
)(_tail_body)


def kernel(users, movies, U, M, W1, b1, W2, b2):
    users = users.astype(jnp.int32)
    movies = movies.astype(jnp.int32)
    w1u = jnp.zeros((HP, NF), jnp.float32).at[:10].set(W1[:, :NF])
    w1m = jnp.zeros((HP, NF), jnp.float32).at[:10].set(W1[:, NF:])
    w2p = jnp.zeros((HP,), jnp.float32).at[:10].set(W2[0])
    b1p = jnp.zeros((HP,), jnp.float32).at[:10].set(b1)
    pm = _project(w1m, M.T, 16384)
    hm = _sc_mgather(movies, *pm)
    pu = _project(w1u, U.T, 65536)
    out = _sc_tail(users, w2p, b1p, b2, *pu, *hm)
    return out


# speedup vs baseline: 6.1999x; 1.0209x over previous
import functools

import jax
import jax.numpy as jnp
from jax import lax
from jax.experimental import pallas as pl
from jax.experimental.pallas import tpu as pltpu
from jax.experimental.pallas import tpu_sc as plsc

BATCH = 16384
NF = 64
HP = 16
NH = 10

_info = plsc.get_sparse_core_info()
_NC, _NS = _info.num_cores, _info.num_subcores
_NW = _NC * _NS
_BPW = BATCH // _NW


def _project_body(lo, w_ref, t_ref, *out_refs):
    wh = w_ref[:, :NF] if lo else w_ref[:, NF:]
    p = jnp.dot(wh, t_ref[:], preferred_element_type=jnp.float32)
    for f, o_ref in enumerate(out_refs):
        o_ref[:] = p[f, :]


def _project(w1, table_t, cols, lo):
    n = table_t.shape[1]
    grid = pl.cdiv(n, cols)
    return pl.pallas_call(
        functools.partial(_project_body, lo),
        grid=(grid,),
        in_specs=[
            pl.BlockSpec((NH, 2 * NF), lambda i: (0, 0)),
            pl.BlockSpec((NF, cols), lambda i: (0, i)),
        ],
        out_specs=[pl.BlockSpec((cols,), lambda i: (i,)) for _ in range(NH)],
        out_shape=[jax.ShapeDtypeStruct((n,), jnp.float32) for _ in range(NH)],
    )(w1, table_t)


def _mgather_body(movies_hbm, *rest):
    pm_hbm = rest[:NH]
    out_hbm = rest[NH:2 * NH]
    idx_m, gath, sem = rest[2 * NH:2 * NH + 3]

    wid = lax.axis_index("s") * _NC + lax.axis_index("c")
    base = wid * _BPW
    pltpu.sync_copy(movies_hbm.at[pl.ds(base, _BPW)], idx_m)
    copies = [pltpu.async_copy(pm_hbm[f].at[idx_m], gath.at[f], sem)
              for f in range(NH)]
    for c in copies:
        c.wait()
    for f in range(NH):
        pltpu.sync_copy(gath.at[f], out_hbm[f].at[pl.ds(base, _BPW)])


_sc_mgather = functools.partial(
    pl.kernel,
    out_type=[jax.ShapeDtypeStruct((BATCH,), jnp.float32) for _ in range(NH)],
    mesh=plsc.VectorSubcoreMesh(core_axis_name="c", subcore_axis_name="s"),
    compiler_params=pltpu.CompilerParams(use_tc_tiling_on_sc=False),
    scratch_types=[
        pltpu.VMEM((_BPW,), jnp.int32),
        pltpu.VMEM((NH, _BPW), jnp.float32),
        pltpu.SemaphoreType.DMA,
    ],
)(_mgather_body)


def _tail_body(users_hbm, w2_hbm, b1_hbm, b2_hbm, *rest):
    pu_hbm = rest[:NH]
    hm_hbm = rest[NH:2 * NH]
    out_hbm = rest[2 * NH]
    idx_u, coef_v, acc_v, gath = rest[2 * NH + 1:2 * NH + 5]
    sem = rest[2 * NH + 5]

    wid = lax.axis_index("s") * _NC + lax.axis_index("c")
    base = wid * _BPW
    pltpu.sync_copy(users_hbm.at[pl.ds(base, _BPW)], idx_u)
    pltpu.sync_copy(w2_hbm, coef_v.at[0])
    pltpu.sync_copy(b1_hbm, coef_v.at[1])
    pltpu.sync_copy(b2_hbm, coef_v.at[2, pl.ds(0, 1)])

    copies = [pltpu.async_copy(pu_hbm[f].at[idx_u], gath.at[0, f], sem)
              for f in range(NH)]
    copies += [pltpu.async_copy(hm_hbm[f].at[pl.ds(base, _BPW)], gath.at[1, f], sem)
               for f in range(NH)]
    for c in copies:
        c.wait()

    w2v = coef_v[0]
    b1v = coef_v[1]
    b2v = coef_v[2]

    def group(g, carry):
        sl = pl.ds(g * 16, 16)
        acc = jnp.zeros((16,), jnp.float32) + b2v[0]
        for f in range(NH):
            h = gath[0, f, sl] + gath[1, f, sl] + b1v[f]
            acc = acc + w2v[f] * jnp.maximum(h, 0.0)
        sig = 1.0 / (1.0 + jnp.exp(-acc))
        acc_v[sl] = sig * 6.0 - 0.5
        return carry

    lax.fori_loop(0, _BPW // 16, group, 0)
    pltpu.sync_copy(acc_v, out_hbm.at[pl.ds(base, _BPW)])


_sc_tail = functools.partial(
    pl.kernel,
    out_type=jax.ShapeDtypeStruct((BATCH,), jnp.float32),
    mesh=plsc.VectorSubcoreMesh(core_axis_name="c", subcore_axis_name="s"),
    compiler_params=pltpu.CompilerParams(use_tc_tiling_on_sc=False),
    scratch_types=[
        pltpu.VMEM((_BPW,), jnp.int32),
        pltpu.VMEM((3, HP), jnp.float32),
        pltpu.VMEM((_BPW,), jnp.float32),
        pltpu.VMEM((2, NH, _BPW), jnp.float32),
        pltpu.SemaphoreType.DMA,
    ],
)(_tail_body)


def kernel(users, movies, U, M, W1, b1, W2, b2):
    users = users.astype(jnp.int32)
    movies = movies.astype(jnp.int32)
    w2p = jnp.zeros((HP,), jnp.float32).at[:10].set(W2[0])
    b1p = jnp.zeros((HP,), jnp.float32).at[:10].set(b1)
    pm = _project(W1, M.T, 32768, False)
    hm = _sc_mgather(movies, *pm)
    pu = _project(W1, U.T, 65536, True)
    out = _sc_tail(users, w2p, b1p, b2, *pu, *hm)
    return out


# speedup vs baseline: 6.7479x; 1.0884x over previous
import functools

import jax
import jax.numpy as jnp
from jax import lax
from jax.experimental import pallas as pl
from jax.experimental.pallas import tpu as pltpu
from jax.experimental.pallas import tpu_sc as plsc

BATCH = 16384
NF = 64
HP = 16
NH = 10
NP = 5

_info = plsc.get_sparse_core_info()
_NC, _NS = _info.num_cores, _info.num_subcores
_NW = _NC * _NS
_BPW = BATCH // _NW


def _project_body(lo, w_ref, t_ref, *out_refs):
    wh = w_ref[:, :NF] if lo else w_ref[:, NF:]
    p = jnp.dot(wh, t_ref[:], preferred_element_type=jnp.float32)
    pb = lax.bitcast_convert_type(p.astype(jnp.bfloat16), jnp.uint16).astype(jnp.uint32)
    for k, o_ref in enumerate(out_refs):
        o_ref[:] = pb[2 * k, :] | (pb[2 * k + 1, :] << 16)


def _project(w1, table_t, cols, lo):
    n = table_t.shape[1]
    grid = pl.cdiv(n, cols)
    return pl.pallas_call(
        functools.partial(_project_body, lo),
        grid=(grid,),
        in_specs=[
            pl.BlockSpec((NH, 2 * NF), lambda i: (0, 0)),
            pl.BlockSpec((NF, cols), lambda i: (0, i)),
        ],
        out_specs=[pl.BlockSpec((cols,), lambda i: (i,)) for _ in range(NP)],
        out_shape=[jax.ShapeDtypeStruct((n,), jnp.uint32) for _ in range(NP)],
    )(w1, table_t)


def _mgather_body(movies_hbm, *rest):
    pm_hbm = rest[:NP]
    out_hbm = rest[NP:2 * NP]
    idx_m, gath, sem = rest[2 * NP:2 * NP + 3]

    wid = lax.axis_index("s") * _NC + lax.axis_index("c")
    base = wid * _BPW
    pltpu.sync_copy(movies_hbm.at[pl.ds(base, _BPW)], idx_m)
    copies = [pltpu.async_copy(pm_hbm[f].at[idx_m], gath.at[f], sem)
              for f in range(NP)]
    for c in copies:
        c.wait()
    for f in range(NP):
        pltpu.sync_copy(gath.at[f], out_hbm[f].at[pl.ds(base, _BPW)])


_sc_mgather = functools.partial(
    pl.kernel,
    out_type=[jax.ShapeDtypeStruct((BATCH,), jnp.uint32) for _ in range(NP)],
    mesh=plsc.VectorSubcoreMesh(core_axis_name="c", subcore_axis_name="s"),
    compiler_params=pltpu.CompilerParams(use_tc_tiling_on_sc=False),
    scratch_types=[
        pltpu.VMEM((_BPW,), jnp.int32),
        pltpu.VMEM((NP, _BPW), jnp.uint32),
        pltpu.SemaphoreType.DMA,
    ],
)(_mgather_body)


def _tail_body(users_hbm, w2_hbm, b1_hbm, b2_hbm, *rest):
    pu_hbm = rest[:NP]
    hm_hbm = rest[NP:2 * NP]
    out_hbm = rest[2 * NP]
    idx_u, coef_v, acc_v, gath = rest[2 * NP + 1:2 * NP + 5]
    sem = rest[2 * NP + 5]

    wid = lax.axis_index("s") * _NC + lax.axis_index("c")
    base = wid * _BPW
    pltpu.sync_copy(users_hbm.at[pl.ds(base, _BPW)], idx_u)
    pltpu.sync_copy(w2_hbm, coef_v.at[0])
    pltpu.sync_copy(b1_hbm, coef_v.at[1])
    pltpu.sync_copy(b2_hbm, coef_v.at[2, pl.ds(0, 1)])

    copies = [pltpu.async_copy(pu_hbm[f].at[idx_u], gath.at[0, f], sem)
              for f in range(NP)]
    copies += [pltpu.async_copy(hm_hbm[f].at[pl.ds(base, _BPW)], gath.at[1, f], sem)
               for f in range(NP)]
    for c in copies:
        c.wait()

    w2v = coef_v[0]
    b1v = coef_v[1]
    b2v = coef_v[2]

    def group(g, carry):
        sl = pl.ds(g * 16, 16)
        acc = jnp.zeros((16,), jnp.float32) + b2v[0]
        for k in range(NP):
            wu = gath[0, k, sl]
            wm = gath[1, k, sl]
            for half in range(2):
                if half == 0:
                    pu_f = plsc.bitcast(wu << 16, jnp.float32)
                    pm_f = plsc.bitcast(wm << 16, jnp.float32)
                else:
                    pu_f = plsc.bitcast(wu & jnp.uint32(0xFFFF0000), jnp.float32)
                    pm_f = plsc.bitcast(wm & jnp.uint32(0xFFFF0000), jnp.float32)
                f = 2 * k + half
                h = pu_f + pm_f + b1v[f]
                acc = acc + w2v[f] * jnp.maximum(h, 0.0)
        sig = 1.0 / (1.0 + jnp.exp(-acc))
        acc_v[sl] = sig * 6.0 - 0.5
        return carry

    lax.fori_loop(0, _BPW // 16, group, 0)
    pltpu.sync_copy(acc_v, out_hbm.at[pl.ds(base, _BPW)])


_sc_tail = functools.partial(
    pl.kernel,
    out_type=jax.ShapeDtypeStruct((BATCH,), jnp.float32),
    mesh=plsc.VectorSubcoreMesh(core_axis_name="c", subcore_axis_name="s"),
    compiler_params=pltpu.CompilerParams(use_tc_tiling_on_sc=False,
                                         needs_layout_passes=False),
    scratch_types=[
        pltpu.VMEM((_BPW,), jnp.int32),
        pltpu.VMEM((3, HP), jnp.float32),
        pltpu.VMEM((_BPW,), jnp.float32),
        pltpu.VMEM((2, NP, _BPW), jnp.uint32),
        pltpu.SemaphoreType.DMA,
    ],
)(_tail_body)


def kernel(users, movies, U, M, W1, b1, W2, b2):
    users = users.astype(jnp.int32)
    movies = movies.astype(jnp.int32)
    w2p = jnp.zeros((HP,), jnp.float32).at[:10].set(W2[0])
    b1p = jnp.zeros((HP,), jnp.float32).at[:10].set(b1)
    pm = _project(W1, M.T, 32768, False)
    hm = _sc_mgather(movies, *pm)
    pu = _project(W1, U.T, 65536, True)
    out = _sc_tail(users, w2p, b1p, b2, *pu, *hm)
    return out
